# baseline (speedup 1.0000x reference)
import functools

import jax
import jax.numpy as jnp
from jax import lax
from jax.experimental import pallas as pl
from jax.experimental.pallas import tpu as pltpu
from jax.experimental.pallas import tpu_sc as plsc

NUM_EMB = 100000
EMBED_DIM = 64
BATCH = 16384

_info = plsc.get_sparse_core_info()
_NC, _NS = _info.num_cores, _info.num_subcores
_NW = _NC * _NS
_B_PER_W = BATCH // _NW
_C = 128
_NCH = _B_PER_W // _C


def _gather_body(y_hbm, emb_hbm, out_hbm, idx_v, buf0, buf1, gsem, ssem):
    wid = lax.axis_index("s") * _NC + lax.axis_index("c")
    base = wid * _B_PER_W
    pltpu.sync_copy(y_hbm.at[pl.ds(base, _B_PER_W)], idx_v)
    bufs = (buf0, buf1)
    gathers = [pltpu.async_copy(emb_hbm.at[idx_v.at[pl.ds(0, _C)]], buf0, gsem)]
    stores = []
    for c in range(_NCH):
        gathers[c].wait()
        stores.append(
            pltpu.async_copy(bufs[c % 2], out_hbm.at[pl.ds(base + c * _C, _C)], ssem)
        )
        if c + 1 < _NCH:
            if c >= 1:
                stores[c - 1].wait()
            gathers.append(
                pltpu.async_copy(
                    emb_hbm.at[idx_v.at[pl.ds((c + 1) * _C, _C)]],
                    bufs[(c + 1) % 2],
                    gsem,
                )
            )
    for s in stores[max(0, _NCH - 2):]:
        s.wait()


@jax.jit
def kernel(y, emb):
    mesh = plsc.VectorSubcoreMesh(core_axis_name="c", subcore_axis_name="s")
    f = functools.partial(
        pl.kernel,
        mesh=mesh,
        out_type=jax.ShapeDtypeStruct((BATCH, EMBED_DIM), jnp.float32),
        scratch_types=[
            pltpu.VMEM((_B_PER_W,), jnp.int32),
            pltpu.VMEM((_C, EMBED_DIM), jnp.float32),
            pltpu.VMEM((_C, EMBED_DIM), jnp.float32),
            pltpu.SemaphoreType.DMA,
            pltpu.SemaphoreType.DMA,
        ],
        compiler_params=pltpu.CompilerParams(use_tc_tiling_on_sc=False),
    )(_gather_body)
    return f(y, emb)


# speedup vs baseline: 1.5222x; 1.5222x over previous
import functools

import jax
import jax.numpy as jnp
from jax import lax
from jax.experimental import pallas as pl
from jax.experimental.pallas import tpu as pltpu
from jax.experimental.pallas import tpu_sc as plsc

NUM_EMB = 100000
EMBED_DIM = 64
BATCH = 16384

_info = plsc.get_sparse_core_info()
_NC, _NS = _info.num_cores, _info.num_subcores
_NW = _NC * _NS
_B_PER_W = BATCH // _NW


def _gather_body(y_hbm, emb_hbm, out_hbm, idx_v, buf, gsem):
    wid = lax.axis_index("s") * _NC + lax.axis_index("c")
    base = wid * _B_PER_W
    pltpu.sync_copy(y_hbm.at[pl.ds(base, _B_PER_W)], idx_v)

    def body(k, _):
        kb = k * 16
        v = idx_v[pl.ds(kb, 16)]
        for l in range(16):
            pltpu.async_copy(emb_hbm.at[v[l]], buf.at[kb + l], gsem)
        return 0

    lax.fori_loop(0, _B_PER_W // 16, body, 0)
    pltpu.make_async_copy(emb_hbm.at[pl.ds(0, _B_PER_W)], buf, gsem).wait()
    pltpu.sync_copy(buf, out_hbm.at[pl.ds(base, _B_PER_W)])


@jax.jit
def kernel(y, emb):
    mesh = plsc.VectorSubcoreMesh(core_axis_name="c", subcore_axis_name="s")
    f = functools.partial(
        pl.kernel,
        mesh=mesh,
        out_type=jax.ShapeDtypeStruct((BATCH, EMBED_DIM), jnp.float32),
        scratch_types=[
            pltpu.VMEM((_B_PER_W,), jnp.int32),
            pltpu.VMEM((_B_PER_W, EMBED_DIM), jnp.float32),
            pltpu.SemaphoreType.DMA,
        ],
    )(_gather_body)
    return f(y, emb)


# speedup vs baseline: 1.7565x; 1.1540x over previous
import functools

import jax
import jax.numpy as jnp
from jax import lax
from jax.experimental import pallas as pl
from jax.experimental.pallas import tpu as pltpu
from jax.experimental.pallas import tpu_sc as plsc

NUM_EMB = 100000
EMBED_DIM = 64
BATCH = 16384

_info = plsc.get_sparse_core_info()
_NC, _NS = _info.num_cores, _info.num_subcores
_NW = _NC * _NS
_B_PER_W = BATCH // _NW
_T_PER_W = _B_PER_W // 8


def _gather_body(y_hbm, emb_hbm, out_hbm, idx_v, buf, gsem):
    wid = lax.axis_index("s") * _NC + lax.axis_index("c")
    base = wid * _B_PER_W
    pltpu.sync_copy(y_hbm.at[pl.ds(base, _B_PER_W)], idx_v)

    def body(k, _):
        v = idx_v[pl.ds(k * 16, 16)]
        for l in range(16):
            idx = v[l]
            q = jax.lax.shift_right_logical(idx, 3)
            r = jax.lax.rem(idx, 8)
            pltpu.async_copy(
                emb_hbm.at[q, r], buf.at[2 * k + l // 8, l % 8], gsem
            )
        return 0

    lax.fori_loop(0, _B_PER_W // 16, body, 0)
    pltpu.make_async_copy(emb_hbm.at[pl.ds(0, _T_PER_W)], buf, gsem).wait()
    pltpu.sync_copy(buf, out_hbm.at[pl.ds(wid * _T_PER_W, _T_PER_W)])


@jax.jit
def kernel(y, emb):
    emb3 = emb.reshape(NUM_EMB // 8, 8, EMBED_DIM)
    mesh = plsc.VectorSubcoreMesh(core_axis_name="c", subcore_axis_name="s")
    f = functools.partial(
        pl.kernel,
        mesh=mesh,
        out_type=jax.ShapeDtypeStruct((BATCH // 8, 8, EMBED_DIM), jnp.float32),
        scratch_types=[
            pltpu.VMEM((_B_PER_W,), jnp.int32),
            pltpu.VMEM((_T_PER_W, 8, EMBED_DIM), jnp.float32),
            pltpu.SemaphoreType.DMA,
        ],
    )(_gather_body)
    out3 = f(y, emb3)
    return out3.reshape(BATCH, EMBED_DIM)
